# baseline (device time: 260129 ns/iter reference)
import jax
import jax.numpy as jnp
from jax import lax
from jax.experimental import pallas as pl
from jax.experimental.pallas import tpu as pltpu

N_DEV = 16
B, SQ, D = 4, 256, 1024
HQ_LOC, DH = 8, 128
KV_LOC = 2
ROWS = B * SQ
CHUNK = ROWS // N_DEV
N_STEPS = 2 * (N_DEV - 1)
SCALE = 0.08838834764831843


def kernel(x, Wq, Wo, Wk, Wv):
    def body(x_ref, wq_ref, wo_ref, wk_hbm, wv_hbm, out_ref,
             wk_s, wv_s, attn_ref, recv_ref,
             local_sems, send_sem, recv_sem, credit_sem):
        d = lax.axis_index("i")
        left = (d + N_DEV - 1) % N_DEV
        right = (d + 1) % N_DEV

        cp_k = pltpu.make_async_copy(
            wk_hbm.at[:, pl.ds(d * KV_LOC * DH, KV_LOC * DH)],
            wk_s, local_sems.at[0])
        cp_v = pltpu.make_async_copy(
            wv_hbm.at[:, pl.ds(d * KV_LOC * DH, KV_LOC * DH)],
            wv_s, local_sems.at[1])
        cp_k.start()
        cp_v.start()

        barrier = pltpu.get_barrier_semaphore()
        for nbr in (left, right):
            pl.semaphore_signal(barrier, inc=1, device_id=(nbr,),
                                device_id_type=pl.DeviceIdType.MESH)
        pl.semaphore_wait(barrier, 2)

        xb = x_ref[...].astype(jnp.bfloat16)
        q_all = jnp.dot(xb, wq_ref[...].astype(jnp.bfloat16),
                        preferred_element_type=jnp.float32
                        ).astype(jnp.bfloat16)
        cp_k.wait()
        cp_v.wait()
        k_all = jnp.dot(xb, wk_s[...].astype(jnp.bfloat16),
                        preferred_element_type=jnp.float32
                        ).astype(jnp.bfloat16)
        v_all = jnp.dot(xb, wv_s[...].astype(jnp.bfloat16),
                        preferred_element_type=jnp.float32
                        ).astype(jnp.bfloat16)

        for b in range(B):
            r0 = b * SQ
            for h in range(HQ_LOC):
                g = h // 4
                q = q_all[r0:r0 + SQ, h * DH:(h + 1) * DH]
                k = k_all[r0:r0 + SQ, g * DH:(g + 1) * DH]
                v = v_all[r0:r0 + SQ, g * DH:(g + 1) * DH]
                s = lax.dot_general(q, k, (((1,), (1,)), ((), ())),
                                    preferred_element_type=jnp.float32)
                s = s * SCALE
                m = jnp.max(s, axis=1, keepdims=True)
                p = jnp.exp(s - m)
                den = jnp.sum(p, axis=1, keepdims=True)
                pn = (p / den).astype(jnp.bfloat16)
                o = jnp.dot(pn, v, preferred_element_type=jnp.float32)
                attn_ref[r0:r0 + SQ, h * DH:(h + 1) * DH] = \
                    o.astype(jnp.bfloat16)

        out_ref[...] = jnp.dot(attn_ref[...],
                               wo_ref[...].astype(jnp.bfloat16),
                               preferred_element_type=jnp.float32)

        for step in range(N_STEPS):
            if step < N_DEV - 1:
                send_c = (d + 2 * N_DEV - step) % N_DEV
                recv_c = (d + 2 * N_DEV - step - 1) % N_DEV
            else:
                t = step - (N_DEV - 1)
                send_c = (d + 2 * N_DEV + 1 - t) % N_DEV
                recv_c = (d + 2 * N_DEV - t) % N_DEV

            if step > 0:
                pl.semaphore_wait(credit_sem, 1)

            src = out_ref.at[pl.ds(send_c * CHUNK, CHUNK), :]
            if step < N_DEV - 1:
                dst = recv_ref
            else:
                dst = out_ref.at[pl.ds(send_c * CHUNK, CHUNK), :]
            rdma = pltpu.make_async_remote_copy(
                src_ref=src, dst_ref=dst,
                send_sem=send_sem, recv_sem=recv_sem,
                device_id=(right,), device_id_type=pl.DeviceIdType.MESH)
            rdma.start()
            rdma.wait_recv()

            if step < N_DEV - 1:
                out_ref[pl.ds(recv_c * CHUNK, CHUNK), :] = (
                    out_ref[pl.ds(recv_c * CHUNK, CHUNK), :]
                    + recv_ref[...])
            if step < N_STEPS - 1:
                pl.semaphore_signal(credit_sem, inc=1, device_id=(left,),
                                    device_id_type=pl.DeviceIdType.MESH)
            rdma.wait_send()

    out = pl.pallas_call(
        body,
        out_shape=jax.ShapeDtypeStruct((ROWS, D), jnp.float32),
        in_specs=[
            pl.BlockSpec(memory_space=pltpu.VMEM),
            pl.BlockSpec(memory_space=pltpu.VMEM),
            pl.BlockSpec(memory_space=pltpu.VMEM),
            pl.BlockSpec(memory_space=pltpu.HBM),
            pl.BlockSpec(memory_space=pltpu.HBM),
        ],
        out_specs=pl.BlockSpec(memory_space=pltpu.VMEM),
        scratch_shapes=[
            pltpu.VMEM((D, KV_LOC * DH), jnp.float32),
            pltpu.VMEM((D, KV_LOC * DH), jnp.float32),
            pltpu.VMEM((ROWS, HQ_LOC * DH), jnp.bfloat16),
            pltpu.VMEM((CHUNK, D), jnp.float32),
            pltpu.SemaphoreType.DMA((2,)),
            pltpu.SemaphoreType.DMA,
            pltpu.SemaphoreType.DMA,
            pltpu.SemaphoreType.REGULAR,
        ],
        compiler_params=pltpu.CompilerParams(collective_id=0),
    )(x.reshape(ROWS, D), Wq, Wo, Wk, Wv)
    return out.reshape(B, SQ, D)


# device time: 67710 ns/iter; 3.8418x vs baseline; 3.8418x over previous
import jax
import jax.numpy as jnp
from jax import lax
from jax.experimental import pallas as pl
from jax.experimental.pallas import tpu as pltpu

N_DEV = 16
B, SQ, D = 4, 256, 1024
HQ_LOC, DH = 8, 128
KV_LOC = 2
ROWS = B * SQ
CHUNK = ROWS // N_DEV
SCALE = 0.08838834764831843

_MESH = pl.DeviceIdType.MESH


def kernel(x, Wq, Wo, Wk, Wv):
    def body(x_ref, wq_ref, wo_ref, wk_hbm, wv_hbm, out_ref,
             wk_s, wv_s, attn_ref, rs_send, rs_stage, ag_send, ag_stage,
             local_sems, rs_ssem, rs_rsem, ag_ssem, ag_rsem):
        d = lax.axis_index("i")

        cp_k = pltpu.make_async_copy(
            wk_hbm.at[:, pl.ds(d * KV_LOC * DH, KV_LOC * DH)],
            wk_s, local_sems.at[0])
        cp_v = pltpu.make_async_copy(
            wv_hbm.at[:, pl.ds(d * KV_LOC * DH, KV_LOC * DH)],
            wv_s, local_sems.at[1])
        cp_k.start()
        cp_v.start()

        barrier = pltpu.get_barrier_semaphore()
        for k in range(1, N_DEV):
            pl.semaphore_signal(barrier, inc=1, device_id=((d + k) % N_DEV,),
                                device_id_type=_MESH)
        pl.semaphore_wait(barrier, N_DEV - 1)

        xb = x_ref[...].astype(jnp.bfloat16)
        q_all = jnp.dot(xb, wq_ref[...].astype(jnp.bfloat16),
                        preferred_element_type=jnp.float32
                        ).astype(jnp.bfloat16)
        cp_k.wait()
        cp_v.wait()
        k_all = jnp.dot(xb, wk_s[...].astype(jnp.bfloat16),
                        preferred_element_type=jnp.float32
                        ).astype(jnp.bfloat16)
        v_all = jnp.dot(xb, wv_s[...].astype(jnp.bfloat16),
                        preferred_element_type=jnp.float32
                        ).astype(jnp.bfloat16)

        for b in range(B):
            r0 = b * SQ
            for h in range(HQ_LOC):
                g = h // 4
                q = q_all[r0:r0 + SQ, h * DH:(h + 1) * DH]
                kk = k_all[r0:r0 + SQ, g * DH:(g + 1) * DH]
                v = v_all[r0:r0 + SQ, g * DH:(g + 1) * DH]
                s = lax.dot_general(q, kk, (((1,), (1,)), ((), ())),
                                    preferred_element_type=jnp.float32)
                s = s * SCALE
                m = jnp.max(s, axis=1, keepdims=True)
                p = jnp.exp(s - m)
                den = jnp.sum(p, axis=1, keepdims=True)
                pn = (p / den).astype(jnp.bfloat16)
                o = jnp.dot(pn, v, preferred_element_type=jnp.float32)
                attn_ref[r0:r0 + SQ, h * DH:(h + 1) * DH] = \
                    o.astype(jnp.bfloat16)

        partial = jnp.dot(attn_ref[...], wo_ref[...].astype(jnp.bfloat16),
                          preferred_element_type=jnp.float32)
        out_ref[...] = partial
        rs_send[...] = partial.astype(jnp.bfloat16)

        rs_desc = []
        for k in range(1, N_DEV):
            dest = (d + k) % N_DEV
            r = pltpu.make_async_remote_copy(
                src_ref=rs_send.at[pl.ds(dest * CHUNK, CHUNK), :],
                dst_ref=rs_stage.at[d],
                send_sem=rs_ssem, recv_sem=rs_rsem,
                device_id=(dest,), device_id_type=_MESH)
            r.start()
            rs_desc.append(r)
        for r in rs_desc:
            r.wait_recv()

        rs_stage[d] = jnp.zeros((CHUNK, D), jnp.bfloat16)
        summed = (out_ref[pl.ds(d * CHUNK, CHUNK), :]
                  + jnp.sum(rs_stage[...].astype(jnp.float32), axis=0))
        out_ref[pl.ds(d * CHUNK, CHUNK), :] = summed
        ag_send[...] = summed.astype(jnp.bfloat16)

        ag_desc = []
        for k in range(1, N_DEV):
            dest = (d + k) % N_DEV
            r = pltpu.make_async_remote_copy(
                src_ref=ag_send, dst_ref=ag_stage.at[d],
                send_sem=ag_ssem, recv_sem=ag_rsem,
                device_id=(dest,), device_id_type=_MESH)
            r.start()
            ag_desc.append(r)
        for r in ag_desc:
            r.wait_recv()

        for k in range(N_DEV):
            @pl.when(k != d)
            def _(k=k):
                out_ref[k * CHUNK:(k + 1) * CHUNK, :] = \
                    ag_stage[k].astype(jnp.float32)

        for r in rs_desc:
            r.wait_send()
        for r in ag_desc:
            r.wait_send()

    out = pl.pallas_call(
        body,
        out_shape=jax.ShapeDtypeStruct((ROWS, D), jnp.float32),
        in_specs=[
            pl.BlockSpec(memory_space=pltpu.VMEM),
            pl.BlockSpec(memory_space=pltpu.VMEM),
            pl.BlockSpec(memory_space=pltpu.VMEM),
            pl.BlockSpec(memory_space=pltpu.HBM),
            pl.BlockSpec(memory_space=pltpu.HBM),
        ],
        out_specs=pl.BlockSpec(memory_space=pltpu.VMEM),
        scratch_shapes=[
            pltpu.VMEM((D, KV_LOC * DH), jnp.float32),
            pltpu.VMEM((D, KV_LOC * DH), jnp.float32),
            pltpu.VMEM((ROWS, HQ_LOC * DH), jnp.bfloat16),
            pltpu.VMEM((ROWS, D), jnp.bfloat16),
            pltpu.VMEM((N_DEV, CHUNK, D), jnp.bfloat16),
            pltpu.VMEM((CHUNK, D), jnp.bfloat16),
            pltpu.VMEM((N_DEV, CHUNK, D), jnp.bfloat16),
            pltpu.SemaphoreType.DMA((2,)),
            pltpu.SemaphoreType.DMA,
            pltpu.SemaphoreType.DMA,
            pltpu.SemaphoreType.DMA,
            pltpu.SemaphoreType.DMA,
        ],
        compiler_params=pltpu.CompilerParams(collective_id=0),
    )(x.reshape(ROWS, D), Wq, Wo, Wk, Wv)
    return out.reshape(B, SQ, D)
